# Initial kernel scaffold; baseline (speedup 1.0000x reference)
import jax
import jax.numpy as jnp
from jax.experimental import pallas as pl


def kernel(X_nucl, X_proto, X_pbs, X_rt, We, Wproto, Wpbs, Wrt):
    raise NotImplementedError("write your pallas kernel here")


# speedup vs baseline: 26.0419x; 26.0419x over previous
import functools

import jax
import jax.numpy as jnp
from jax.experimental import pallas as pl

EMBED = 128
SEQ = 200


def _lookup_body(xn_ref, xp_ref, xpb_ref, xrt_ref, wcat_ref, out_ref):
    bs = xn_ref.shape[0]
    k = jax.lax.broadcasted_iota(jnp.int32, (bs, SEQ, EMBED), 2)
    tgt = jnp.where(k < 5, xn_ref[...][:, :, None],
          jnp.where(k < 8, xp_ref[...][:, :, None] + 5,
          jnp.where(k < 11, xpb_ref[...][:, :, None] + 8,
                    xrt_ref[...][:, :, None] + 11)))
    onehot = (k == tgt).astype(jnp.float32).reshape(bs * SEQ, EMBED)
    res = jnp.dot(onehot, wcat_ref[...], preferred_element_type=jnp.float32)
    out_ref[...] = res.reshape(bs, SEQ, EMBED)


@jax.jit
def kernel(X_nucl, X_proto, X_pbs, X_rt, We, Wproto, Wpbs, Wrt):
    B, L = X_nucl.shape
    wcat = jnp.zeros((EMBED, EMBED), jnp.float32)
    wcat = wcat.at[0:5].set(We).at[5:8].set(Wproto)
    wcat = wcat.at[8:11].set(Wpbs).at[11:14].set(Wrt)

    bs = 64
    grid = (B // bs,)
    idx_spec = pl.BlockSpec((bs, L), lambda i: (i, 0))
    return pl.pallas_call(
        _lookup_body,
        grid=grid,
        in_specs=[idx_spec, idx_spec, idx_spec, idx_spec,
                  pl.BlockSpec((EMBED, EMBED), lambda i: (0, 0))],
        out_specs=pl.BlockSpec((bs, L, EMBED), lambda i: (i, 0, 0)),
        out_shape=jax.ShapeDtypeStruct((B, L, EMBED), jnp.float32),
    )(X_nucl.astype(jnp.int32), X_proto.astype(jnp.int32),
      X_pbs.astype(jnp.int32), X_rt.astype(jnp.int32), wcat)


# speedup vs baseline: 70.7217x; 2.7157x over previous
import functools

import jax
import jax.numpy as jnp
from jax.experimental import pallas as pl

EMBED = 128
SEQ = 200


def _lookup_body(xn_ref, xp_ref, xpb_ref, xrt_ref, wcat_ref, out_ref):
    bs = xn_ref.shape[0]
    j = jax.lax.broadcasted_iota(jnp.int32, (EMBED, EMBED), 0)
    k = jax.lax.broadcasted_iota(jnp.int32, (EMBED, EMBED), 1)
    r = j - 5
    g = jnp.where((j < 5) & (k == j), 1.0, 0.0)
    ann = ((k == 5 + r // 9).astype(jnp.float32)
           + (k == 8 + (r // 3) % 3).astype(jnp.float32)
           + (k == 11 + r % 3).astype(jnp.float32))
    g = g + jnp.where((j >= 5) & (j < 32), ann, 0.0)
    fused = jnp.dot(g, wcat_ref[...], preferred_element_type=jnp.float32)

    c2 = (xn_ref[...] * 32
          + (xp_ref[...] * 3 + xpb_ref[...]) * 3 + xrt_ref[...])
    cb = jax.lax.broadcast_in_dim(c2, (bs, SEQ, EMBED), (0, 1))
    kl = jax.lax.broadcasted_iota(jnp.int32, (bs, SEQ, EMBED), 2)
    onehot = ((kl == (cb >> 5)) | ((kl - 5) == (cb & 31))
              ).astype(jnp.float32).reshape(bs * SEQ, EMBED)
    res = jnp.dot(onehot, fused, preferred_element_type=jnp.float32)
    out_ref[...] = res.reshape(bs, SEQ, EMBED)


@jax.jit
def kernel(X_nucl, X_proto, X_pbs, X_rt, We, Wproto, Wpbs, Wrt):
    B, L = X_nucl.shape
    wcat = jnp.zeros((EMBED, EMBED), jnp.float32)
    wcat = wcat.at[0:5].set(We).at[5:8].set(Wproto)
    wcat = wcat.at[8:11].set(Wpbs).at[11:14].set(Wrt)

    bs = 64
    grid = (B // bs,)
    idx_spec = pl.BlockSpec((bs, L), lambda i: (i, 0))
    return pl.pallas_call(
        _lookup_body,
        grid=grid,
        in_specs=[idx_spec, idx_spec, idx_spec, idx_spec,
                  pl.BlockSpec((EMBED, EMBED), lambda i: (0, 0))],
        out_specs=pl.BlockSpec((bs, L, EMBED), lambda i: (i, 0, 0)),
        out_shape=jax.ShapeDtypeStruct((B, L, EMBED), jnp.float32),
    )(X_nucl.astype(jnp.int32), X_proto.astype(jnp.int32),
      X_pbs.astype(jnp.int32), X_rt.astype(jnp.int32), wcat)


# speedup vs baseline: 74.1673x; 1.0487x over previous
import functools

import jax
import jax.numpy as jnp
from jax.experimental import pallas as pl

EMBED = 128
SEQ = 200


def _lookup_body(xn_ref, xp_ref, xpb_ref, xrt_ref, wcat_ref, out_ref):
    bs = xn_ref.shape[0]
    j = jax.lax.broadcasted_iota(jnp.int32, (EMBED, EMBED), 0)
    k = jax.lax.broadcasted_iota(jnp.int32, (EMBED, EMBED), 1)
    r = j - 5
    g = jnp.where((j < 5) & (k == j), 1.0, 0.0)
    ann = ((k == 5 + r // 9).astype(jnp.float32)
           + (k == 8 + (r // 3) % 3).astype(jnp.float32)
           + (k == 11 + r % 3).astype(jnp.float32))
    g = g + jnp.where((j >= 5) & (j < 32), ann, 0.0)
    fused = jnp.dot(g, wcat_ref[...], preferred_element_type=jnp.float32)

    c2 = (xn_ref[...] * 32
          + (xp_ref[...] * 3 + xpb_ref[...]) * 3 + xrt_ref[...])
    cb = jax.lax.broadcast_in_dim(c2, (bs, SEQ, EMBED), (0, 1))
    kl = jax.lax.broadcasted_iota(jnp.int32, (bs, SEQ, EMBED), 2)
    onehot = ((kl == (cb >> 5)) | ((kl - 5) == (cb & 31))
              ).astype(jnp.float32).reshape(bs * SEQ, EMBED)
    res = jnp.dot(onehot, fused, preferred_element_type=jnp.float32)
    out_ref[...] = res.reshape(bs, SEQ, EMBED)


@jax.jit
def kernel(X_nucl, X_proto, X_pbs, X_rt, We, Wproto, Wpbs, Wrt):
    B, L = X_nucl.shape
    wcat = jnp.zeros((EMBED, EMBED), jnp.float32)
    wcat = wcat.at[0:5].set(We).at[5:8].set(Wproto)
    wcat = wcat.at[8:11].set(Wpbs).at[11:14].set(Wrt)

    bs = 128
    grid = (B // bs,)
    idx_spec = pl.BlockSpec((bs, L), lambda i: (i, 0))
    return pl.pallas_call(
        _lookup_body,
        grid=grid,
        in_specs=[idx_spec, idx_spec, idx_spec, idx_spec,
                  pl.BlockSpec((EMBED, EMBED), lambda i: (0, 0))],
        out_specs=pl.BlockSpec((bs, L, EMBED), lambda i: (i, 0, 0)),
        out_shape=jax.ShapeDtypeStruct((B, L, EMBED), jnp.float32),
    )(X_nucl.astype(jnp.int32), X_proto.astype(jnp.int32),
      X_pbs.astype(jnp.int32), X_rt.astype(jnp.int32), wcat)


# speedup vs baseline: 74.8315x; 1.0090x over previous
import functools

import jax
import jax.numpy as jnp
from jax.experimental import pallas as pl

EMBED = 128
SEQ = 200


def _lookup_body(xn_ref, xp_ref, xpb_ref, xrt_ref, wcat_ref, out_ref):
    bs = xn_ref.shape[0]
    j = jax.lax.broadcasted_iota(jnp.int32, (EMBED, EMBED), 0)
    k = jax.lax.broadcasted_iota(jnp.int32, (EMBED, EMBED), 1)
    r = j - 5
    g = jnp.where((j < 5) & (k == j), 1.0, 0.0)
    ann = ((k == 5 + r // 9).astype(jnp.float32)
           + (k == 8 + (r // 3) % 3).astype(jnp.float32)
           + (k == 11 + r % 3).astype(jnp.float32))
    g = g + jnp.where((j >= 5) & (j < 32), ann, 0.0)
    fused = jnp.dot(g, wcat_ref[...], preferred_element_type=jnp.float32)

    c2 = (xn_ref[...] * 32
          + (xp_ref[...] * 3 + xpb_ref[...]) * 3 + xrt_ref[...])
    cb = jax.lax.broadcast_in_dim(c2, (bs, SEQ, EMBED), (0, 1))
    kl = jax.lax.broadcasted_iota(jnp.int32, (bs, SEQ, EMBED), 2)
    onehot = ((kl == (cb >> 5)) | ((kl - 5) == (cb & 31))
              ).astype(jnp.float32).reshape(bs * SEQ, EMBED)
    res = jnp.dot(onehot, fused, preferred_element_type=jnp.float32)
    out_ref[...] = res.reshape(bs, SEQ, EMBED)


@jax.jit
def kernel(X_nucl, X_proto, X_pbs, X_rt, We, Wproto, Wpbs, Wrt):
    B, L = X_nucl.shape
    wcat = jnp.zeros((EMBED, EMBED), jnp.float32)
    wcat = wcat.at[0:5].set(We).at[5:8].set(Wproto)
    wcat = wcat.at[8:11].set(Wpbs).at[11:14].set(Wrt)

    bs = 256
    grid = (B // bs,)
    idx_spec = pl.BlockSpec((bs, L), lambda i: (i, 0))
    return pl.pallas_call(
        _lookup_body,
        grid=grid,
        in_specs=[idx_spec, idx_spec, idx_spec, idx_spec,
                  pl.BlockSpec((EMBED, EMBED), lambda i: (0, 0))],
        out_specs=pl.BlockSpec((bs, L, EMBED), lambda i: (i, 0, 0)),
        out_shape=jax.ShapeDtypeStruct((B, L, EMBED), jnp.float32),
    )(X_nucl.astype(jnp.int32), X_proto.astype(jnp.int32),
      X_pbs.astype(jnp.int32), X_rt.astype(jnp.int32), wcat)
